# baseline (speedup 1.0000x reference)
import functools

import jax
import jax.numpy as jnp
from jax import lax
from jax.experimental import pallas as pl
from jax.experimental.pallas import tpu as pltpu
from jax.experimental.pallas import tpu_sc as plsc

VOCAB = 1000000
HIDDEN = 64
OUT = 2
B = 4096
L = 200

NC = 2
NS = 16
NW = NC * NS
SPW = B // NW
LANES = 16
SGRP = SPW // LANES

VBLK = 65536
VGRID = -(-VOCAB // VBLK)
QPAD = VGRID * VBLK
CHUNK = QPAD // NS


def _proj_body(w_ref, tabT_ref, q_ref):
    p = jnp.dot(w_ref[...], tabT_ref[...],
                preferred_element_type=jnp.float32) * (1.0 / L)
    lo = lax.bitcast_convert_type(p[0].astype(jnp.bfloat16), jnp.uint16)
    hi = lax.bitcast_convert_type(p[1].astype(jnp.bfloat16), jnp.uint16)
    word = hi.astype(jnp.uint32) << 16 | lo.astype(jnp.uint32)
    q_ref[...] = lax.bitcast_convert_type(word, jnp.int32)


_proj = pl.pallas_call(
    _proj_body,
    grid=(VGRID,),
    in_specs=[
        pl.BlockSpec((OUT, HIDDEN), lambda i: (0, 0)),
        pl.BlockSpec((HIDDEN, VBLK), lambda i: (0, i)),
    ],
    out_specs=pl.BlockSpec((VBLK,), lambda i: (i,)),
    out_shape=jax.ShapeDtypeStruct((QPAD,), jnp.int32),
)


def _sc_body(textT_ref, q_ref, b_ref, out_ref, idx_v, buf, b_v, out_v,
             qs_v, sem0, sem1, semt):
    c = lax.axis_index("c")
    s = lax.axis_index("s")
    wid = s * NC + c
    base = wid * SPW

    pltpu.sync_copy(q_ref.at[pl.ds(s * CHUNK, CHUNK)],
                    qs_v.at[pl.ds(s * CHUNK, CHUNK)])
    pltpu.sync_copy(textT_ref.at[:, pl.ds(base, SPW)], idx_v)
    pltpu.sync_copy(b_ref, b_v)
    plsc.subcore_barrier()

    sems = (sem0, sem1)

    def fire(t, parity):
        pltpu.async_copy(qs_v.at[idx_v.at[t]], buf.at[parity], sems[parity])

    def drain(parity):
        pltpu.make_async_copy(qs_v.at[idx_v.at[0]], buf.at[parity],
                              sems[parity]).wait()

    fire(0, 0)
    fire(1, 1)

    zero = jnp.zeros((LANES,), jnp.float32)

    def pair_body(p, accs):
        accs = list(accs)
        for parity in range(2):
            t = 2 * p + parity
            drain(parity)
            new = []
            for g in range(SGRP):
                w32 = buf[parity, pl.ds(g * LANES, LANES)]
                pair = plsc.bitcast(w32, jnp.bfloat16)
                a, b = plsc.unpack(pair, format=plsc.PackFormat.INTERLEAVED)
                new.append(accs[2 * g] + a)
                new.append(accs[2 * g + 1] + b)
            accs = new

            nxt = t + 2

            @pl.when(nxt < L)
            def _():
                fire(nxt, parity)

        return tuple(accs)

    accs = lax.fori_loop(0, L // 2, pair_body, (zero,) * (2 * SGRP))

    lane = lax.iota(jnp.int32, LANES)
    b0 = b_v[pl.ds(0, LANES)]
    b1 = b_v[pl.ds(LANES, LANES)]
    for g in range(SGRP):
        pos = (lane + g * LANES) * OUT
        plsc.store_scatter(out_v, [pos], accs[2 * g] + b0)
        plsc.store_scatter(out_v, [pos + 1], accs[2 * g + 1] + b1)

    pltpu.sync_copy(out_v, out_ref.at[pl.ds(OUT * base, OUT * SPW)])


@functools.partial(
    pl.kernel,
    out_type=jax.ShapeDtypeStruct((B * OUT,), jnp.float32),
    mesh=plsc.VectorSubcoreMesh(core_axis_name="c", subcore_axis_name="s",
                                num_cores=NC, num_subcores=NS),
    scratch_types=[
        pltpu.VMEM((L, SPW), jnp.int32),
        pltpu.VMEM((2, SPW), jnp.int32),
        pltpu.VMEM((2 * LANES,), jnp.float32),
        pltpu.VMEM((OUT * SPW,), jnp.float32),
        pltpu.VMEM_SHARED((QPAD,), jnp.int32),
        pltpu.SemaphoreType.DMA,
        pltpu.SemaphoreType.DMA,
        pltpu.SemaphoreType.DMA,
    ],
    compiler_params=pltpu.CompilerParams(use_tc_tiling_on_sc=False,
                                         needs_layout_passes=False),
)
def _sc_sums(textT_ref, q_ref, b_ref, out_ref, idx_v, buf, b_v, out_v,
             qs_v, sem0, sem1, semt):
    _sc_body(textT_ref, q_ref, b_ref, out_ref, idx_v, buf, b_v, out_v,
             qs_v, sem0, sem1, semt)


def kernel(text, emb_table, fc1_w, fc1_b):
    tabT = emb_table.T
    textT = text.astype(jnp.int32).T
    q = _proj(fc1_w, tabT)
    b32 = jnp.concatenate([jnp.full((LANES,), fc1_b[0], jnp.float32),
                           jnp.full((LANES,), fc1_b[1], jnp.float32)])
    out = _sc_sums(textT, q, b32)
    return out.reshape(B, OUT)


# speedup vs baseline: 1.0833x; 1.0833x over previous
import functools

import jax
import jax.numpy as jnp
from jax import lax
from jax.experimental import pallas as pl
from jax.experimental.pallas import tpu as pltpu
from jax.experimental.pallas import tpu_sc as plsc

VOCAB = 1000000
HIDDEN = 64
OUT = 2
B = 4096
L = 200

NC = 2
NS = 16
NW = NC * NS
SPW = B // NW
LANES = 16
SGRP = SPW // LANES

VBLK = 32768
VGRID = -(-VOCAB // VBLK)
QPAD = VGRID * VBLK
CHUNK = QPAD // NS


def _proj_body(w_ref, tabT_ref, q_ref):
    p = jnp.dot(w_ref[...], tabT_ref[...],
                preferred_element_type=jnp.float32) * (1.0 / L)
    lo = lax.bitcast_convert_type(p[0].astype(jnp.bfloat16), jnp.uint16)
    hi = lax.bitcast_convert_type(p[1].astype(jnp.bfloat16), jnp.uint16)
    word = hi.astype(jnp.uint32) << 16 | lo.astype(jnp.uint32)
    q_ref[...] = lax.bitcast_convert_type(word, jnp.int32)


_proj = pl.pallas_call(
    _proj_body,
    grid=(VGRID,),
    in_specs=[
        pl.BlockSpec((OUT, HIDDEN), lambda i: (0, 0)),
        pl.BlockSpec((HIDDEN, VBLK), lambda i: (0, i)),
    ],
    out_specs=pl.BlockSpec((VBLK,), lambda i: (i,)),
    out_shape=jax.ShapeDtypeStruct((QPAD,), jnp.int32),
)


def _sc_body(textT_ref, q_ref, b_ref, out_ref, idx_v, buf, b_v, out_v,
             qs_v, sem0, sem1, sem2, sem3):
    c = lax.axis_index("c")
    s = lax.axis_index("s")
    wid = s * NC + c
    base = wid * SPW

    pltpu.sync_copy(q_ref.at[pl.ds(s * CHUNK, CHUNK)],
                    qs_v.at[pl.ds(s * CHUNK, CHUNK)])
    pltpu.sync_copy(textT_ref.at[:, pl.ds(base, SPW)], idx_v)
    pltpu.sync_copy(b_ref, b_v)
    plsc.subcore_barrier()

    sems = (sem0, sem1, sem2, sem3)
    NBUF = 4

    def fire(t, slot):
        pltpu.async_copy(qs_v.at[idx_v.at[t]], buf.at[slot], sems[slot])

    def drain(slot):
        pltpu.make_async_copy(qs_v.at[idx_v.at[0]], buf.at[slot],
                              sems[slot]).wait()

    for j in range(NBUF):
        fire(j, j)

    zero = jnp.zeros((LANES,), jnp.float32)

    def quad_body(p, accs):
        accs = list(accs)
        for j in range(NBUF):
            t = NBUF * p + j
            drain(j)
            new = []
            for g in range(SGRP):
                w32 = buf[j, pl.ds(g * LANES, LANES)]
                pair = plsc.bitcast(w32, jnp.bfloat16)
                a, b = plsc.unpack(pair, format=plsc.PackFormat.INTERLEAVED)
                new.append(accs[2 * g] + a)
                new.append(accs[2 * g + 1] + b)
            accs = new

            nxt = t + NBUF

            @pl.when(nxt < L)
            def _():
                fire(nxt, j)

        return tuple(accs)

    accs = lax.fori_loop(0, L // NBUF, quad_body, (zero,) * (2 * SGRP))

    lane = lax.iota(jnp.int32, LANES)
    b0 = b_v[pl.ds(0, LANES)]
    b1 = b_v[pl.ds(LANES, LANES)]
    for g in range(SGRP):
        pos = (lane + g * LANES) * OUT
        plsc.store_scatter(out_v, [pos], accs[2 * g] + b0)
        plsc.store_scatter(out_v, [pos + 1], accs[2 * g + 1] + b1)

    pltpu.sync_copy(out_v, out_ref.at[pl.ds(OUT * base, OUT * SPW)])


@functools.partial(
    pl.kernel,
    out_type=jax.ShapeDtypeStruct((B * OUT,), jnp.float32),
    mesh=plsc.VectorSubcoreMesh(core_axis_name="c", subcore_axis_name="s",
                                num_cores=NC, num_subcores=NS),
    scratch_types=[
        pltpu.VMEM((L, SPW), jnp.int32),
        pltpu.VMEM((4, SPW), jnp.int32),
        pltpu.VMEM((2 * LANES,), jnp.float32),
        pltpu.VMEM((OUT * SPW,), jnp.float32),
        pltpu.VMEM_SHARED((QPAD,), jnp.int32),
        pltpu.SemaphoreType.DMA,
        pltpu.SemaphoreType.DMA,
        pltpu.SemaphoreType.DMA,
        pltpu.SemaphoreType.DMA,
    ],
    compiler_params=pltpu.CompilerParams(use_tc_tiling_on_sc=False,
                                         needs_layout_passes=False),
)
def _sc_sums(textT_ref, q_ref, b_ref, out_ref, idx_v, buf, b_v, out_v,
             qs_v, sem0, sem1, sem2, sem3):
    _sc_body(textT_ref, q_ref, b_ref, out_ref, idx_v, buf, b_v, out_v,
             qs_v, sem0, sem1, sem2, sem3)


def kernel(text, emb_table, fc1_w, fc1_b):
    tabT = emb_table.T
    textT = text.astype(jnp.int32).T
    q = _proj(fc1_w, tabT)
    b32 = jnp.concatenate([jnp.full((LANES,), fc1_b[0], jnp.float32),
                           jnp.full((LANES,), fc1_b[1], jnp.float32)])
    out = _sc_sums(textT, q, b32)
    return out.reshape(B, OUT)


# speedup vs baseline: 1.0982x; 1.0138x over previous
import functools

import jax
import jax.numpy as jnp
from jax import lax
from jax.experimental import pallas as pl
from jax.experimental.pallas import tpu as pltpu
from jax.experimental.pallas import tpu_sc as plsc

VOCAB = 1000000
HIDDEN = 64
OUT = 2
B = 4096
L = 200

NC = 2
NS = 16
NW = NC * NS
SPW = B // NW
LANES = 16
SGRP = SPW // LANES

VBLK = 32768
VGRID = -(-VOCAB // VBLK)
QPAD = VGRID * VBLK
CHUNK = QPAD // NS


def _proj_body(w_ref, tabT_ref, q_ref):
    p = jnp.dot(w_ref[...], tabT_ref[...],
                preferred_element_type=jnp.float32) * (1.0 / L)
    lo = lax.bitcast_convert_type(p[0].astype(jnp.bfloat16), jnp.uint16)
    hi = lax.bitcast_convert_type(p[1].astype(jnp.bfloat16), jnp.uint16)
    word = hi.astype(jnp.uint32) << 16 | lo.astype(jnp.uint32)
    q_ref[...] = lax.bitcast_convert_type(word, jnp.int32)


_proj = pl.pallas_call(
    _proj_body,
    grid=(VGRID,),
    in_specs=[
        pl.BlockSpec((OUT, HIDDEN), lambda i: (0, 0)),
        pl.BlockSpec((HIDDEN, VBLK), lambda i: (0, i)),
    ],
    out_specs=pl.BlockSpec((VBLK,), lambda i: (i,)),
    out_shape=jax.ShapeDtypeStruct((QPAD,), jnp.int32),
)


def _sc_body(textT_ref, q_ref, b_ref, out_ref, idx_v, buf, b_v, out_v,
             qs_v, sem0, sem1, sem2, sem3, sem4, sem5, sem6, sem7):
    c = lax.axis_index("c")
    s = lax.axis_index("s")
    wid = s * NC + c
    base = wid * SPW

    pltpu.sync_copy(q_ref.at[pl.ds(s * CHUNK, CHUNK)],
                    qs_v.at[pl.ds(s * CHUNK, CHUNK)])
    pltpu.sync_copy(textT_ref.at[:, pl.ds(base, SPW)], idx_v)
    pltpu.sync_copy(b_ref, b_v)
    plsc.subcore_barrier()

    sems = (sem0, sem1, sem2, sem3, sem4, sem5, sem6, sem7)
    NBUF = 8

    def fire(t, slot):
        pltpu.async_copy(qs_v.at[idx_v.at[t]], buf.at[slot], sems[slot])

    def drain(slot):
        pltpu.make_async_copy(qs_v.at[idx_v.at[0]], buf.at[slot],
                              sems[slot]).wait()

    for j in range(NBUF):
        fire(j, j)

    zero = jnp.zeros((LANES,), jnp.float32)

    def quad_body(p, accs):
        accs = list(accs)
        for j in range(NBUF):
            t = NBUF * p + j
            drain(j)
            new = []
            for g in range(SGRP):
                w32 = buf[j, pl.ds(g * LANES, LANES)]
                pair = plsc.bitcast(w32, jnp.bfloat16)
                a, b = plsc.unpack(pair, format=plsc.PackFormat.INTERLEAVED)
                new.append(accs[2 * g] + a)
                new.append(accs[2 * g + 1] + b)
            accs = new

            nxt = t + NBUF

            @pl.when(nxt < L)
            def _():
                fire(nxt, j)

        return tuple(accs)

    accs = lax.fori_loop(0, L // NBUF, quad_body, (zero,) * (2 * SGRP))

    lane = lax.iota(jnp.int32, LANES)
    b0 = b_v[pl.ds(0, LANES)]
    b1 = b_v[pl.ds(LANES, LANES)]
    for g in range(SGRP):
        pos = (lane + g * LANES) * OUT
        plsc.store_scatter(out_v, [pos], accs[2 * g] + b0)
        plsc.store_scatter(out_v, [pos + 1], accs[2 * g + 1] + b1)

    pltpu.sync_copy(out_v, out_ref.at[pl.ds(OUT * base, OUT * SPW)])


@functools.partial(
    pl.kernel,
    out_type=jax.ShapeDtypeStruct((B * OUT,), jnp.float32),
    mesh=plsc.VectorSubcoreMesh(core_axis_name="c", subcore_axis_name="s",
                                num_cores=NC, num_subcores=NS),
    scratch_types=[
        pltpu.VMEM((L, SPW), jnp.int32),
        pltpu.VMEM((8, SPW), jnp.int32),
        pltpu.VMEM((2 * LANES,), jnp.float32),
        pltpu.VMEM((OUT * SPW,), jnp.float32),
        pltpu.VMEM_SHARED((QPAD,), jnp.int32),
        pltpu.SemaphoreType.DMA,
        pltpu.SemaphoreType.DMA,
        pltpu.SemaphoreType.DMA,
        pltpu.SemaphoreType.DMA,
        pltpu.SemaphoreType.DMA,
        pltpu.SemaphoreType.DMA,
        pltpu.SemaphoreType.DMA,
        pltpu.SemaphoreType.DMA,
    ],
    compiler_params=pltpu.CompilerParams(use_tc_tiling_on_sc=False,
                                         needs_layout_passes=False),
)
def _sc_sums(textT_ref, q_ref, b_ref, out_ref, idx_v, buf, b_v, out_v,
             qs_v, sem0, sem1, sem2, sem3, sem4, sem5, sem6, sem7):
    _sc_body(textT_ref, q_ref, b_ref, out_ref, idx_v, buf, b_v, out_v,
             qs_v, sem0, sem1, sem2, sem3, sem4, sem5, sem6, sem7)


def kernel(text, emb_table, fc1_w, fc1_b):
    tabT = emb_table.T
    textT = text.astype(jnp.int32).T
    q = _proj(fc1_w, tabT)
    b32 = jnp.concatenate([jnp.full((LANES,), fc1_b[0], jnp.float32),
                           jnp.full((LANES,), fc1_b[1], jnp.float32)])
    out = _sc_sums(textT, q, b32)
    return out.reshape(B, OUT)
